# baseline (speedup 1.0000x reference)
import functools

import jax
import jax.numpy as jnp
from jax import lax
from jax.experimental import pallas as pl
from jax.experimental.pallas import tpu as pltpu
from jax.experimental.pallas import tpu_sc as plsc

B, S, D, E, K = 4, 8192, 768, 8, 2
N = B * S
NCHUNK = 2
NH = N // NCHUNK
T = 4096
NC, NS, L = 2, 16, 16
NW = NC * NS
TOK_W = NH // NW


def _router_block(x_ref, wt_ref, weights_t_ref):
    x = x_ref[...]
    wt = wt_ref[...]
    logits_t = lax.dot_general(
        wt, x, (((1,), (1,)), ((), ())), preferred_element_type=jnp.float32
    )
    m1 = jnp.max(logits_t, axis=0, keepdims=True)
    ex = jnp.exp(logits_t - m1)
    weights_t_ref[...] = ex / jnp.sum(ex, axis=0, keepdims=True)


def _tc_softmax(n):
    return pl.pallas_call(
        _router_block,
        grid=(n // T,),
        in_specs=[
            pl.BlockSpec((T, D), lambda i: (i, 0)),
            pl.BlockSpec((E, D), lambda i: (0, 0)),
        ],
        out_specs=pl.BlockSpec((E, T), lambda i: (0, i)),
        out_shape=jax.ShapeDtypeStruct((E, n), jnp.float32),
    )


@functools.partial(
    pl.kernel,
    mesh=plsc.VectorSubcoreMesh(core_axis_name="c", subcore_axis_name="s"),
    out_type=[
        jax.ShapeDtypeStruct((K, NH), jnp.float32),
        jax.ShapeDtypeStruct((K, NH), jnp.int32),
    ],
    scratch_types=[
        pltpu.VMEM((E, TOK_W), jnp.float32),
        pltpu.VMEM((K, TOK_W), jnp.float32),
        pltpu.VMEM((K, TOK_W), jnp.int32),
    ],
)
def _sc_top2(w_hbm, topw_hbm, tope_hbm, w_v, topw_v, tope_v):
    wid = lax.axis_index("s") * NC + lax.axis_index("c")
    base = wid * TOK_W
    pltpu.sync_copy(w_hbm.at[:, pl.ds(base, TOK_W)], w_v)

    def body(g, carry):
        col = g * L
        rows = [w_v[e, pl.ds(col, L)] for e in range(E)]
        m1 = rows[0]
        id1 = jnp.zeros((L,), jnp.int32)
        for e in range(1, E):
            gt = rows[e] > m1
            m1 = jnp.where(gt, rows[e], m1)
            id1 = jnp.where(gt, jnp.int32(e), id1)
        m2 = jnp.full((L,), -1.0, jnp.float32)
        id2 = jnp.zeros((L,), jnp.int32)
        for e in range(E):
            take = (rows[e] > m2) & (id1 != jnp.int32(e))
            m2 = jnp.where(take, rows[e], m2)
            id2 = jnp.where(take, jnp.int32(e), id2)
        ssum = m1 + m2
        topw_v[0, pl.ds(col, L)] = m1 / ssum
        topw_v[1, pl.ds(col, L)] = m2 / ssum
        tope_v[0, pl.ds(col, L)] = id1
        tope_v[1, pl.ds(col, L)] = id2
        return carry

    lax.fori_loop(0, TOK_W // L, body, 0)
    pltpu.sync_copy(topw_v, topw_hbm.at[:, pl.ds(base, TOK_W)])
    pltpu.sync_copy(tope_v, tope_hbm.at[:, pl.ds(base, TOK_W)])


@jax.jit
def kernel(x, W):
    xf = x.reshape(N, D)
    wt = W.T
    w_halves = [_tc_softmax(NH)(xf[h * NH:(h + 1) * NH], wt)
                for h in range(NCHUNK)]
    top_halves = [_sc_top2(wh) for wh in w_halves]
    weights_t = jnp.concatenate(w_halves, axis=1)
    topw_t = jnp.concatenate([t[0] for t in top_halves], axis=1)
    tope_t = jnp.concatenate([t[1] for t in top_halves], axis=1)
    return (
        weights_t.T.reshape(B, S, E),
        topw_t.T.reshape(B, S, K),
        tope_t.T.reshape(B, S, K),
    )


# speedup vs baseline: 3.4317x; 3.4317x over previous
import jax
import jax.numpy as jnp
from jax import lax
from jax.experimental import pallas as pl
from jax.experimental.pallas import tpu as pltpu

B, S, D, E, K = 4, 8192, 768, 8, 2
N = B * S

_SIZES = [512, 512, 1024, 2048] + [4096] * 6 + [2048, 1024, 512, 512]
assert sum(_SIZES) == N
_SIZE_CLASSES = [512, 1024, 2048, 4096]
_RING = 2

_ENTRIES = []
_use_count = {sz: 0 for sz in _SIZE_CLASSES}
_prev_user = {}
_off = 0
for _i, _sz in enumerate(_SIZES):
    _slot = _use_count[_sz] % _RING
    _use_count[_sz] += 1
    _buf = (_SIZE_CLASSES.index(_sz), _slot)
    _dep = _prev_user.get(_buf, -1)
    _prev_user[_buf] = _i
    _ENTRIES.append([_off, _sz, _buf, _dep])
    _off += _sz
_ENTRIES[12][3] = 8
_ENTRIES[13][3] = 9


def _top2_softmax(logits_t, weights_slice, topw_slice, tope_slice):
    ids = lax.broadcasted_iota(jnp.int32, logits_t.shape, 0)
    big = jnp.int32(E)
    neg = jnp.float32(-jnp.inf)

    m1 = jnp.max(logits_t, axis=0, keepdims=True)
    id1 = jnp.min(jnp.where(logits_t == m1, ids, big), axis=0, keepdims=True)
    l2 = jnp.where(ids == id1, neg, logits_t)
    m2 = jnp.max(l2, axis=0, keepdims=True)
    id2 = jnp.min(jnp.where(l2 == m2, ids, big), axis=0, keepdims=True)

    ex = jnp.exp(logits_t - m1)
    weights_slice[...] = ex / jnp.sum(ex, axis=0, keepdims=True)

    e2 = jnp.exp(m2 - m1)
    w1 = 1.0 / (1.0 + e2)
    topw_slice[...] = jnp.concatenate([w1, 1.0 - w1], axis=0)
    tope_slice[...] = jnp.concatenate([id1, id2], axis=0)


def _router_manual(x_hbm, wt_ref, weights_ref, topw_ref, tope_ref,
                   *scratch):
    bufs = {}
    idx = 0
    for ci, sz in enumerate(_SIZE_CLASSES):
        for slot in range(_RING):
            bufs[(ci, slot)] = scratch[idx]
            idx += 1
    sems = scratch[idx]

    wt = wt_ref[...]

    def start(i):
        off, sz, buf, _ = _ENTRIES[i]
        pltpu.make_async_copy(
            x_hbm.at[pl.ds(off, sz), :], bufs[buf], sems.at[i]
        ).start()

    for i, (_, _, _, dep) in enumerate(_ENTRIES):
        if dep == -1:
            start(i)

    for i, (off, sz, buf, _) in enumerate(_ENTRIES):
        pltpu.make_async_copy(
            x_hbm.at[pl.ds(off, sz), :], bufs[buf], sems.at[i]
        ).wait()
        logits_t = lax.dot_general(
            wt, bufs[buf][...], (((1,), (1,)), ((), ())),
            preferred_element_type=jnp.float32,
        )
        _top2_softmax(
            logits_t,
            weights_ref.at[:, pl.ds(off, sz)],
            topw_ref.at[:, pl.ds(off, sz)],
            tope_ref.at[:, pl.ds(off, sz)],
        )
        for j, (_, _, _, dep) in enumerate(_ENTRIES):
            if dep == i:
                start(j)


@jax.jit
def kernel(x, W):
    xf = x.reshape(N, D)
    wt = W.T
    scratch_shapes = [
        pltpu.VMEM((sz, D), jnp.float32)
        for sz in _SIZE_CLASSES for _ in range(_RING)
    ] + [pltpu.SemaphoreType.DMA((len(_ENTRIES),))]
    weights_t, topw_t, tope_t = pl.pallas_call(
        _router_manual,
        in_specs=[
            pl.BlockSpec(memory_space=pl.ANY),
            pl.BlockSpec((E, D), lambda: (0, 0)),
        ],
        out_specs=[
            pl.BlockSpec((E, N), lambda: (0, 0)),
            pl.BlockSpec((K, N), lambda: (0, 0)),
            pl.BlockSpec((K, N), lambda: (0, 0)),
        ],
        out_shape=[
            jax.ShapeDtypeStruct((E, N), jnp.float32),
            jax.ShapeDtypeStruct((K, N), jnp.float32),
            jax.ShapeDtypeStruct((K, N), jnp.int32),
        ],
        scratch_shapes=scratch_shapes,
    )(xf, wt)
    return (
        weights_t.T.reshape(B, S, E),
        topw_t.T.reshape(B, S, K),
        tope_t.T.reshape(B, S, K),
    )


# speedup vs baseline: 3.4629x; 1.0091x over previous
import jax
import jax.numpy as jnp
from jax import lax
from jax.experimental import pallas as pl
from jax.experimental.pallas import tpu as pltpu

B, S, D, E, K = 4, 8192, 768, 8, 2
N = B * S

_SIZES = [4096] * 8
assert sum(_SIZES) == N
_SIZE_CLASSES = [4096]
_RING = 3

_ENTRIES = []
_use_count = {sz: 0 for sz in _SIZE_CLASSES}
_prev_user = {}
_off = 0
for _i, _sz in enumerate(_SIZES):
    _slot = _use_count[_sz] % _RING
    _use_count[_sz] += 1
    _buf = (_SIZE_CLASSES.index(_sz), _slot)
    _dep = _prev_user.get(_buf, -1)
    _prev_user[_buf] = _i
    _ENTRIES.append([_off, _sz, _buf, _dep])
    _off += _sz


def _top2_softmax(logits_t, weights_slice, topw_slice, tope_slice):
    ids = lax.broadcasted_iota(jnp.int32, logits_t.shape, 0)
    big = jnp.int32(E)
    neg = jnp.float32(-jnp.inf)

    m1 = jnp.max(logits_t, axis=0, keepdims=True)
    id1 = jnp.min(jnp.where(logits_t == m1, ids, big), axis=0, keepdims=True)
    l2 = jnp.where(ids == id1, neg, logits_t)
    m2 = jnp.max(l2, axis=0, keepdims=True)
    id2 = jnp.min(jnp.where(l2 == m2, ids, big), axis=0, keepdims=True)

    ex = jnp.exp(logits_t - m1)
    weights_slice[...] = ex / jnp.sum(ex, axis=0, keepdims=True)

    e2 = jnp.exp(m2 - m1)
    w1 = 1.0 / (1.0 + e2)
    topw_slice[...] = jnp.concatenate([w1, 1.0 - w1], axis=0)
    tope_slice[...] = jnp.concatenate([id1, id2], axis=0)


def _router_manual(x_hbm, wt_ref, weights_ref, topw_ref, tope_ref,
                   *scratch):
    bufs = {}
    idx = 0
    for ci, sz in enumerate(_SIZE_CLASSES):
        for slot in range(_RING):
            bufs[(ci, slot)] = scratch[idx]
            idx += 1
    sems = scratch[idx]

    wt = wt_ref[...]

    def start(i):
        off, sz, buf, _ = _ENTRIES[i]
        pltpu.make_async_copy(
            x_hbm.at[pl.ds(off, sz), :], bufs[buf], sems.at[i]
        ).start()

    for i, (_, _, _, dep) in enumerate(_ENTRIES):
        if dep == -1:
            start(i)

    for i, (off, sz, buf, _) in enumerate(_ENTRIES):
        pltpu.make_async_copy(
            x_hbm.at[pl.ds(off, sz), :], bufs[buf], sems.at[i]
        ).wait()
        logits_t = lax.dot_general(
            wt, bufs[buf][...], (((1,), (1,)), ((), ())),
            preferred_element_type=jnp.float32,
        )
        _top2_softmax(
            logits_t,
            weights_ref.at[:, pl.ds(off, sz)],
            topw_ref.at[:, pl.ds(off, sz)],
            tope_ref.at[:, pl.ds(off, sz)],
        )
        for j, (_, _, _, dep) in enumerate(_ENTRIES):
            if dep == i:
                start(j)


@jax.jit
def kernel(x, W):
    xf = x.reshape(N, D)
    wt = W.T
    scratch_shapes = [
        pltpu.VMEM((sz, D), jnp.float32)
        for sz in _SIZE_CLASSES for _ in range(_RING)
    ] + [pltpu.SemaphoreType.DMA((len(_ENTRIES),))]
    weights_t, topw_t, tope_t = pl.pallas_call(
        _router_manual,
        in_specs=[
            pl.BlockSpec(memory_space=pl.ANY),
            pl.BlockSpec((E, D), lambda: (0, 0)),
        ],
        out_specs=[
            pl.BlockSpec((E, N), lambda: (0, 0)),
            pl.BlockSpec((K, N), lambda: (0, 0)),
            pl.BlockSpec((K, N), lambda: (0, 0)),
        ],
        out_shape=[
            jax.ShapeDtypeStruct((E, N), jnp.float32),
            jax.ShapeDtypeStruct((K, N), jnp.float32),
            jax.ShapeDtypeStruct((K, N), jnp.int32),
        ],
        scratch_shapes=scratch_shapes,
    )(xf, wt)
    return (
        weights_t.T.reshape(B, S, E),
        topw_t.T.reshape(B, S, K),
        tope_t.T.reshape(B, S, K),
    )


# speedup vs baseline: 3.6430x; 1.0520x over previous
import jax
import jax.numpy as jnp
from jax.experimental import pallas as pl

B, S, D, E, K = 4, 8192, 768, 8, 2
T = 4096


def _router_block(x_ref, wt_ref, weights_t_ref, topw_t_ref, tope_t_ref):
    x = x_ref[...]
    wt = wt_ref[...]
    logits_t = jax.lax.dot_general(
        wt, x, (((1,), (1,)), ((), ())), preferred_element_type=jnp.float32
    )

    ids = jax.lax.broadcasted_iota(jnp.int32, logits_t.shape, 0)
    big = jnp.int32(E)
    neg = jnp.float32(-jnp.inf)

    m1 = jnp.max(logits_t, axis=0, keepdims=True)
    id1 = jnp.min(jnp.where(logits_t == m1, ids, big), axis=0, keepdims=True)
    l2 = jnp.where(ids == id1, neg, logits_t)
    m2 = jnp.max(l2, axis=0, keepdims=True)
    id2 = jnp.min(jnp.where(l2 == m2, ids, big), axis=0, keepdims=True)

    ex = jnp.exp(logits_t - m1)
    denom = jnp.sum(ex, axis=0, keepdims=True)
    weights_t_ref[...] = ex / denom

    e2 = jnp.exp(m2 - m1)
    w1 = 1.0 / (1.0 + e2)
    topw_t_ref[...] = jnp.concatenate([w1, 1.0 - w1], axis=0)
    tope_t_ref[...] = jnp.concatenate([id1, id2], axis=0)


@jax.jit
def kernel(x, W):
    N = B * S
    xf = x.reshape(N, D)
    wt = W.T
    grid = (N // T,)
    weights_t, topw_t, tope_t = pl.pallas_call(
        _router_block,
        grid=grid,
        in_specs=[
            pl.BlockSpec((T, D), lambda i: (i, 0)),
            pl.BlockSpec((E, D), lambda i: (0, 0)),
        ],
        out_specs=[
            pl.BlockSpec((E, T), lambda i: (0, i)),
            pl.BlockSpec((K, T), lambda i: (0, i)),
            pl.BlockSpec((K, T), lambda i: (0, i)),
        ],
        out_shape=[
            jax.ShapeDtypeStruct((E, N), jnp.float32),
            jax.ShapeDtypeStruct((K, N), jnp.float32),
            jax.ShapeDtypeStruct((K, N), jnp.int32),
        ],
    )(xf, wt)
    return (
        weights_t.T.reshape(B, S, E),
        topw_t.T.reshape(B, S, K),
        tope_t.T.reshape(B, S, K),
    )
